# baseline (speedup 1.0000x reference)
import functools

import jax
import jax.numpy as jnp
from jax import lax
from jax.experimental import pallas as pl
from jax.experimental.pallas import tpu as pltpu
from jax.experimental.pallas import tpu_sc as plsc

_B = 4
_N = 5000
_MAXK = 300
_KPAD = 320
_L = 16
_IOU_THR = 0.6
_SCORE_THR = 0.2
_MIN_SIZE = 8.0
_SENT = 2.0e9
_PK_W = 40064
_ORD_W = 5120
_OUT_W = 1280


_CS = 50
_NBLK = _N // _CS


def _nms_body(pk_hbm, ord_hbm, out_hbm,
              pk_v, ord_v, kx1, ky1, kx2, ky2, ka, outf, state):
    cid = lax.axis_index("c")
    sid = lax.axis_index("s")
    wid = cid * 16 + sid

    @pl.when(wid < _B)
    def _():
        b = wid
        pltpu.sync_copy(pk_hbm.at[b], pk_v)
        pltpu.sync_copy(ord_hbm.at[b], ord_v)

        sent = jnp.full((_L,), _SENT, jnp.float32)
        zero = jnp.zeros((_L,), jnp.float32)
        for t in range(_KPAD // _L):
            sl = pl.ds(t * _L, _L)
            kx1[sl] = sent
            ky1[sl] = zero
            kx2[sl] = sent
            ky2[sl] = zero
            ka[sl] = zero

        lane = lax.iota(jnp.int32, 16)
        lanef = lane.astype(jnp.float32)
        ind = [jnp.maximum(1.0 - jnp.abs(lanef - k), 0.0) for k in range(4)]
        indc0 = 1.0 - ind[0]

        def write_box(slot, x1, y1, x2, y2):
            outf[pl.ds(slot * 4, _L)] = (x1 * ind[0] + y1 * ind[1]
                                         + x2 * ind[2] + y2 * ind[3])

        state[0] = 0
        state[1] = 1

        def candidate(j):
            @pl.when(state[1] == 1)
            def _():
                idx = ord_v[pl.ds(j, _L)][0]
                bv = pk_v[pl.ds(idx * 8, _L)]
                x1 = bv[0]
                y1 = bv[1]
                x2 = bv[2]
                y2 = bv[3]
                s = bv[4]
                w = x2 - x1
                h = y2 - y1
                ok = (s > _SCORE_THR) & (w >= _MIN_SIZE) & (h >= _MIN_SIZE)
                aj = w * h
                kc = state[0]
                nch = (kc + (_L - 1)) // _L

                def chunk_body(t, acc):
                    sl = pl.ds(t * _L, _L)
                    xi1 = jnp.maximum(kx1[sl], x1)
                    yi1 = jnp.maximum(ky1[sl], y1)
                    xi2 = jnp.minimum(kx2[sl], x2)
                    yi2 = jnp.minimum(ky2[sl], y2)
                    iw = jnp.maximum(xi2 - xi1, 0.0)
                    ih = jnp.maximum(yi2 - yi1, 0.0)
                    inter = iw * ih
                    return jnp.maximum(acc,
                                       inter - _IOU_THR * (aj + ka[sl] - inter))

                acc = lax.fori_loop(0, nch, chunk_body,
                                    jnp.full((_L,), -1.0, jnp.float32))
                for sh in (8, 4, 2, 1):
                    perm = (lane + sh) % _L
                    acc = jnp.maximum(
                        acc, acc.at[perm].get(mode="promise_in_bounds"))

                keepf = ok & (acc[0] < 0.0)
                sx1 = jnp.where(keepf, x1, _SENT)
                sy1 = jnp.where(keepf, y1, 0.0)
                sx2 = jnp.where(keepf, x2, _SENT)
                sy2 = jnp.where(keepf, y2, 0.0)
                sa = jnp.where(keepf, aj, 0.0)
                sl = pl.ds(kc, _L)
                kx1[sl] = sx1 * ind[0] + _SENT * indc0
                ky1[sl] = sy1 * ind[0]
                kx2[sl] = sx2 * ind[0] + _SENT * indc0
                ky2[sl] = sy2 * ind[0]
                ka[sl] = sa * ind[0]
                write_box(kc, x1, y1, x2, y2)
                kc2 = kc + jnp.where(keepf, 1, 0)
                state[0] = kc2
                state[1] = jnp.where(ok & (kc2 < _MAXK), 1, 0)

        def block(b0, carry):
            @pl.when(state[1] == 1)
            def _():
                def inner(jj, c2):
                    candidate(b0 * _CS + jj)
                    return c2

                lax.fori_loop(0, _CS, inner, 0)
            return carry

        lax.fori_loop(0, _NBLK, block, 0)

        kc = state[0]

        pv = pk_v[pl.ds(ord_v[pl.ds(_N - 1, _L)][0] * 8, _L)]

        def pad_body(r, carry):
            write_box(r, pv[0], pv[1], pv[2], pv[3])
            return carry

        lax.fori_loop(kc, _MAXK, pad_body, 0)

        pltpu.sync_copy(outf, out_hbm.at[b])


_nms_call = functools.partial(
    pl.kernel,
    out_type=jax.ShapeDtypeStruct((_B, _OUT_W), jnp.float32),
    mesh=plsc.VectorSubcoreMesh(core_axis_name="c", subcore_axis_name="s"),
    scratch_types=[
        pltpu.VMEM((_PK_W,), jnp.float32),
        pltpu.VMEM((_ORD_W,), jnp.int32),
        pltpu.VMEM((_KPAD,), jnp.float32),
        pltpu.VMEM((_KPAD,), jnp.float32),
        pltpu.VMEM((_KPAD,), jnp.float32),
        pltpu.VMEM((_KPAD,), jnp.float32),
        pltpu.VMEM((_KPAD,), jnp.float32),
        pltpu.VMEM((_OUT_W,), jnp.float32),
        pltpu.SMEM((8,), jnp.int32),
    ],
)(_nms_body)


@jax.jit
def kernel(proposals, cls_scores):
    scores = cls_scores[:, :, 1]
    w = proposals[:, :, 2] - proposals[:, :, 0]
    h = proposals[:, :, 3] - proposals[:, :, 1]
    valid = (scores > _SCORE_THR) & (w >= _MIN_SIZE) & (h >= _MIN_SIZE)
    s = jnp.where(valid, scores, -jnp.inf)
    order = jnp.argsort(-s, axis=-1).astype(jnp.int32)
    packed = jnp.concatenate(
        [proposals, scores[:, :, None],
         jnp.zeros((_B, _N, 3), jnp.float32)], axis=-1).reshape(_B, _N * 8)
    packed = jnp.concatenate(
        [packed, jnp.zeros((_B, _PK_W - _N * 8), jnp.float32)], axis=-1)
    order = jnp.concatenate(
        [order, jnp.zeros((_B, _ORD_W - _N), jnp.int32)], axis=-1)
    out = _nms_call(packed, order)
    return out[:, :_MAXK * 4].reshape(_B, _MAXK, 4)


# speedup vs baseline: 1.0744x; 1.0744x over previous
import functools

import jax
import jax.numpy as jnp
from jax import lax
from jax.experimental import pallas as pl
from jax.experimental.pallas import tpu as pltpu
from jax.experimental.pallas import tpu_sc as plsc

_B = 4
_N = 5000
_MAXK = 300
_KPAD = 320
_L = 16
_IOU_THR = 0.6
_SCORE_THR = 0.2
_MIN_SIZE = 8.0
_SENT = 2.0e9
_PK_W = 40064
_ORD_W = 5120
_OUT_W = 1280


_CS = 50
_NBLK = _N // _CS


def _nms_body(pk_hbm, ord_hbm, out_hbm,
              pk_v, ord_v, kx1, ky1, kx2, ky2, ka, outf, state):
    cid = lax.axis_index("c")
    sid = lax.axis_index("s")
    wid = cid * 16 + sid

    @pl.when(wid < _B)
    def _():
        b = wid
        pltpu.sync_copy(pk_hbm.at[b], pk_v)
        pltpu.sync_copy(ord_hbm.at[b], ord_v)

        sent = jnp.full((_L,), _SENT, jnp.float32)
        zero = jnp.zeros((_L,), jnp.float32)
        for t in range(_KPAD // _L):
            sl = pl.ds(t * _L, _L)
            kx1[sl] = sent
            ky1[sl] = zero
            kx2[sl] = sent
            ky2[sl] = zero
            ka[sl] = zero

        lane = lax.iota(jnp.int32, 16)
        lanef = lane.astype(jnp.float32)
        ind = [jnp.maximum(1.0 - jnp.abs(lanef - k), 0.0) for k in range(4)]
        indc0 = 1.0 - ind[0]

        def write_box(slot, x1, y1, x2, y2):
            outf[pl.ds(slot * 4, _L)] = (x1 * ind[0] + y1 * ind[1]
                                         + x2 * ind[2] + y2 * ind[3])

        state[0] = 0
        state[1] = 1

        def candidate(j):
            @pl.when(state[1] == 1)
            def _():
                idx = ord_v[pl.ds(j, _L)][0]
                bv = pk_v[pl.ds(idx * 8, _L)]
                x1 = bv[0]
                y1 = bv[1]
                x2 = bv[2]
                y2 = bv[3]
                s = bv[4]
                w = x2 - x1
                h = y2 - y1
                ok = (s > _SCORE_THR) & (w >= _MIN_SIZE) & (h >= _MIN_SIZE)
                aj = w * h
                kc = state[0]
                nch = (kc + (_L - 1)) // _L

                def chunk_body(t, acc):
                    sl = pl.ds(t * _L, _L)
                    xi1 = jnp.maximum(kx1[sl], x1)
                    yi1 = jnp.maximum(ky1[sl], y1)
                    xi2 = jnp.minimum(kx2[sl], x2)
                    yi2 = jnp.minimum(ky2[sl], y2)
                    iw = jnp.maximum(xi2 - xi1, 0.0)
                    ih = jnp.maximum(yi2 - yi1, 0.0)
                    inter = iw * ih
                    return jnp.maximum(acc,
                                       inter - _IOU_THR * (aj + ka[sl] - inter))

                acc = lax.fori_loop(0, nch, chunk_body,
                                    jnp.full((_L,), -1.0, jnp.float32))
                for sh in (8, 4, 2, 1):
                    perm = (lane + sh) % _L
                    acc = jnp.maximum(
                        acc, acc.at[perm].get(mode="promise_in_bounds"))

                keepf = ok & (acc[0] < 0.0)
                sx1 = jnp.where(keepf, x1, _SENT)
                sy1 = jnp.where(keepf, y1, 0.0)
                sx2 = jnp.where(keepf, x2, _SENT)
                sy2 = jnp.where(keepf, y2, 0.0)
                sa = jnp.where(keepf, aj, 0.0)
                sl = pl.ds(kc, _L)
                kx1[sl] = sx1 * ind[0] + _SENT * indc0
                ky1[sl] = sy1 * ind[0]
                kx2[sl] = sx2 * ind[0] + _SENT * indc0
                ky2[sl] = sy2 * ind[0]
                ka[sl] = sa * ind[0]
                write_box(kc, x1, y1, x2, y2)
                kc2 = kc + jnp.where(keepf, 1, 0)
                state[0] = kc2
                state[1] = jnp.where(ok & (kc2 < _MAXK), 1, 0)

        def block(b0, carry):
            @pl.when(state[1] == 1)
            def _():
                def inner(jj, c2):
                    candidate(b0 * _CS + jj)
                    return c2

                lax.fori_loop(0, _CS, inner, 0)
            return carry

        lax.fori_loop(0, _NBLK, block, 0)

        kc = state[0]

        pv = pk_v[pl.ds(ord_v[pl.ds(_N - 1, _L)][0] * 8, _L)]

        def pad_body(r, carry):
            write_box(r, pv[0], pv[1], pv[2], pv[3])
            return carry

        lax.fori_loop(kc, _MAXK, pad_body, 0)

        pltpu.sync_copy(outf, out_hbm.at[b])


_nms_call = functools.partial(
    pl.kernel,
    out_type=jax.ShapeDtypeStruct((_B, _OUT_W), jnp.float32),
    mesh=plsc.VectorSubcoreMesh(core_axis_name="c", subcore_axis_name="s"),
    scratch_types=[
        pltpu.VMEM((_PK_W,), jnp.float32),
        pltpu.VMEM((_ORD_W,), jnp.int32),
        pltpu.VMEM((_KPAD,), jnp.float32),
        pltpu.VMEM((_KPAD,), jnp.float32),
        pltpu.VMEM((_KPAD,), jnp.float32),
        pltpu.VMEM((_KPAD,), jnp.float32),
        pltpu.VMEM((_KPAD,), jnp.float32),
        pltpu.VMEM((_OUT_W,), jnp.float32),
        pltpu.SMEM((8,), jnp.int32),
    ],
)(_nms_body)


@jax.jit
def kernel(proposals, cls_scores):
    scores = cls_scores[:, :, 1]
    w = proposals[:, :, 2] - proposals[:, :, 0]
    h = proposals[:, :, 3] - proposals[:, :, 1]
    valid = (scores > _SCORE_THR) & (w >= _MIN_SIZE) & (h >= _MIN_SIZE)
    s = jnp.where(valid, scores, -jnp.inf)
    m = lax.bitcast_convert_type(s, jnp.int32)
    key = jnp.where(m < 0, m ^ jnp.int32(0x7FFFFFFF), m)
    order = jnp.argsort(-key, axis=-1).astype(jnp.int32)
    packed = jnp.concatenate(
        [proposals, scores[:, :, None],
         jnp.zeros((_B, _N, 3), jnp.float32)], axis=-1).reshape(_B, _N * 8)
    packed = jnp.concatenate(
        [packed, jnp.zeros((_B, _PK_W - _N * 8), jnp.float32)], axis=-1)
    order = jnp.concatenate(
        [order, jnp.zeros((_B, _ORD_W - _N), jnp.int32)], axis=-1)
    out = _nms_call(packed, order)
    return out[:, :_MAXK * 4].reshape(_B, _MAXK, 4)
